# Initial kernel scaffold; baseline (speedup 1.0000x reference)
import functools

import jax
import jax.numpy as jnp
from jax import lax
from jax.experimental import pallas as pl
from jax.experimental.pallas import tpu as pltpu
from jax.experimental.pallas import tpu_sc as plsc

N_NODES = 10000
D_FEAT = 128
N_EDGES = 320000
R_DEC = 2.0
T_DEC = 1.0

_info = plsc.get_sparse_core_info()
NC, NS, L = _info.num_cores, _info.num_subcores, _info.num_lanes
NW = NC * NS
EPW = N_EDGES // NW
B = 80
NB = EPW // B
UNROLL = 8


def _normalize_body(h_ref, o_ref):
    x = h_ref[...]
    n2 = jnp.sum(x * x, axis=1, keepdims=True)
    norm = jnp.sqrt(n2)
    scale = jnp.minimum(jnp.float32(1.0),
                        jnp.float32(1.0) / jnp.maximum(norm, jnp.float32(1e-12)))
    o_ref[...] = x * scale


def _normalize(h):
    rb = 1000
    return pl.pallas_call(
        _normalize_body,
        out_shape=jax.ShapeDtypeStruct((N_NODES, D_FEAT), jnp.float32),
        grid=(N_NODES // rb,),
        in_specs=[pl.BlockSpec((rb, D_FEAT), lambda i: (i, 0))],
        out_specs=pl.BlockSpec((rb, D_FEAT), lambda i: (i, 0)),
    )(h)


@functools.partial(
    pl.kernel,
    out_type=jax.ShapeDtypeStruct((N_EDGES,), jnp.float32),
    mesh=plsc.VectorSubcoreMesh(core_axis_name="c", subcore_axis_name="s"),
    scratch_types=[
        pltpu.VMEM((EPW,), jnp.int32),
        pltpu.VMEM((EPW,), jnp.int32),
        pltpu.VMEM((EPW,), jnp.float32),
        pltpu.VMEM((B, D_FEAT), jnp.float32),
        pltpu.VMEM((B, D_FEAT), jnp.float32),
        pltpu.VMEM((B, D_FEAT), jnp.float32),
        pltpu.VMEM((B, D_FEAT), jnp.float32),
        pltpu.SemaphoreType.DMA,
        pltpu.SemaphoreType.DMA,
    ],
)
def _sc_decode(h_hbm, ei_hbm, out_hbm, idx_i, idx_j, out_v,
               xi0, xj0, xi1, xj1, sem0, sem1):
    wid = lax.axis_index("s") * NC + lax.axis_index("c")
    base = pl.multiple_of(wid * EPW, 8)
    pltpu.sync_copy(ei_hbm.at[pl.ds(base, EPW)], idx_i)
    pltpu.sync_copy(ei_hbm.at[pl.ds(N_EDGES + base, EPW)], idx_j)

    bufs = ((xi0, xj0, sem0), (xi1, xj1, sem1))

    def start(blk, b):
        xi, xj, sem = bufs[b]
        off = pl.multiple_of(blk * B, 8)
        pltpu.async_copy(h_hbm.at[idx_i.at[pl.ds(off, B)]], xi, sem)
        pltpu.async_copy(h_hbm.at[idx_j.at[pl.ds(off, B)]], xj, sem)

    def wait(blk, b):
        xi, xj, sem = bufs[b]
        off = pl.multiple_of(blk * B, 8)
        pltpu.make_async_copy(h_hbm.at[idx_i.at[pl.ds(off, B)]], xi, sem).wait()
        pltpu.make_async_copy(h_hbm.at[idx_j.at[pl.ds(off, B)]], xj, sem).wait()

    def compute(blk, b):
        xi, xj, _ = bufs[b]
        for g in range(B // L):
            rows = jnp.int32(g * L) + lax.iota(jnp.int32, L)
            acc0 = jnp.zeros((L,), jnp.float32)

            def dstep(d, acc):
                for u in range(UNROLL):
                    col = jnp.full((L,), d * UNROLL + u, jnp.int32)
                    a = plsc.load_gather(xi, [rows, col])
                    c = plsc.load_gather(xj, [rows, col])
                    t = a - c
                    acc = acc + t * t
                return acc

            acc = lax.fori_loop(0, D_FEAT // UNROLL, dstep, acc0)
            probs = jnp.float32(1.0) / (
                jnp.exp((acc - jnp.float32(R_DEC)) / jnp.float32(T_DEC))
                + jnp.float32(1.0))
            out_v[pl.ds(pl.multiple_of(blk * B + g * L, 8), L)] = probs

    start(0, 0)
    start(1, 1)

    def pair(p, carry):
        g = p * 2
        wait(g, 0)
        compute(g, 0)

        @pl.when(g + 2 < NB)
        def _():
            start(g + 2, 0)

        wait(g + 1, 1)
        compute(g + 1, 1)

        @pl.when(g + 3 < NB)
        def _():
            start(g + 3, 1)

        return carry

    lax.fori_loop(0, (NB - 1) // 2, pair, 0)
    wait(NB - 1, 0)
    compute(NB - 1, 0)

    pltpu.sync_copy(out_v, out_hbm.at[pl.ds(base, EPW)])


def kernel(h, edge_index):
    h_norm = _normalize(h)
    ei_flat = edge_index.astype(jnp.int32).reshape(-1)
    return _sc_decode(h_norm, ei_flat)


# speedup vs baseline: 1.3416x; 1.3416x over previous
import functools

import jax
import jax.numpy as jnp
from jax import lax
from jax.experimental import pallas as pl
from jax.experimental.pallas import tpu as pltpu
from jax.experimental.pallas import tpu_sc as plsc

N_NODES = 10000
D_FEAT = 128
N_EDGES = 320000
R_DEC = 2.0
T_DEC = 1.0

_info = plsc.get_sparse_core_info()
NC, NS, L = _info.num_cores, _info.num_subcores, _info.num_lanes
NW = NC * NS
EPW = N_EDGES // NW
B = 80
NB = EPW // B
UNROLL = 8


def _normalize_body(h_ref, o_ref):
    x = h_ref[...]
    n2 = jnp.sum(x * x, axis=1, keepdims=True)
    norm = jnp.sqrt(n2)
    scale = jnp.minimum(jnp.float32(1.0),
                        jnp.float32(1.0) / jnp.maximum(norm, jnp.float32(1e-12)))
    o_ref[...] = x * scale


def _normalize(h):
    rb = 1000
    return pl.pallas_call(
        _normalize_body,
        out_shape=jax.ShapeDtypeStruct((N_NODES, D_FEAT), jnp.float32),
        grid=(N_NODES // rb,),
        in_specs=[pl.BlockSpec((rb, D_FEAT), lambda i: (i, 0))],
        out_specs=pl.BlockSpec((rb, D_FEAT), lambda i: (i, 0)),
    )(h)


@functools.partial(
    pl.kernel,
    out_type=jax.ShapeDtypeStruct((N_EDGES,), jnp.float32),
    mesh=plsc.VectorSubcoreMesh(core_axis_name="c", subcore_axis_name="s"),
    compiler_params=pltpu.CompilerParams(needs_layout_passes=False),
    scratch_types=[
        pltpu.VMEM((EPW,), jnp.int32),
        pltpu.VMEM((EPW,), jnp.int32),
        pltpu.VMEM((EPW,), jnp.float32),
        pltpu.VMEM((B, D_FEAT), jnp.float32),
        pltpu.VMEM((B, D_FEAT), jnp.float32),
        pltpu.VMEM((B, D_FEAT), jnp.float32),
        pltpu.VMEM((B, D_FEAT), jnp.float32),
        pltpu.SemaphoreType.DMA,
        pltpu.SemaphoreType.DMA,
    ],
)
def _sc_decode(h_hbm, ei_hbm, out_hbm, idx_i, idx_j, out_v,
               xi0, xj0, xi1, xj1, sem0, sem1):
    wid = lax.axis_index("s") * NC + lax.axis_index("c")
    base = pl.multiple_of(wid * EPW, 8)
    pltpu.sync_copy(ei_hbm.at[pl.ds(base, EPW)], idx_i)
    pltpu.sync_copy(ei_hbm.at[pl.ds(N_EDGES + base, EPW)], idx_j)

    bufs = ((xi0, xj0, sem0), (xi1, xj1, sem1))

    def start(blk, b):
        xi, xj, sem = bufs[b]
        off = pl.multiple_of(blk * B, 8)
        pltpu.async_copy(h_hbm.at[idx_i.at[pl.ds(off, B)]], xi, sem)
        pltpu.async_copy(h_hbm.at[idx_j.at[pl.ds(off, B)]], xj, sem)

    def wait(blk, b):
        xi, xj, sem = bufs[b]
        off = pl.multiple_of(blk * B, 8)
        pltpu.make_async_copy(h_hbm.at[idx_i.at[pl.ds(off, B)]], xi, sem).wait()
        pltpu.make_async_copy(h_hbm.at[idx_j.at[pl.ds(off, B)]], xj, sem).wait()

    def compute(blk, b):
        xi, xj, _ = bufs[b]
        for g in range(B // L):
            rows = jnp.int32(g * L) + lax.iota(jnp.int32, L)
            acc0 = jnp.zeros((L,), jnp.float32)

            def dstep(d, acc):
                for u in range(UNROLL):
                    col = jnp.full((L,), d * UNROLL + u, jnp.int32)
                    a = plsc.load_gather(xi, [rows, col])
                    c = plsc.load_gather(xj, [rows, col])
                    t = a - c
                    acc = acc + t * t
                return acc

            acc = lax.fori_loop(0, D_FEAT // UNROLL, dstep, acc0)
            probs = jnp.float32(1.0) / (
                jnp.exp((acc - jnp.float32(R_DEC)) / jnp.float32(T_DEC))
                + jnp.float32(1.0))
            out_v[pl.ds(pl.multiple_of(blk * B + g * L, 8), L)] = probs

    start(0, 0)
    start(1, 1)

    def pair(p, carry):
        g = p * 2
        wait(g, 0)
        compute(g, 0)

        @pl.when(g + 2 < NB)
        def _():
            start(g + 2, 0)

        wait(g + 1, 1)
        compute(g + 1, 1)

        @pl.when(g + 3 < NB)
        def _():
            start(g + 3, 1)

        return carry

    lax.fori_loop(0, (NB - 1) // 2, pair, 0)
    wait(NB - 1, 0)
    compute(NB - 1, 0)

    pltpu.sync_copy(out_v, out_hbm.at[pl.ds(base, EPW)])


def kernel(h, edge_index):
    h_norm = _normalize(h)
    ei_flat = edge_index.astype(jnp.int32).reshape(-1)
    return _sc_decode(h_norm, ei_flat)


# speedup vs baseline: 8.9079x; 6.6396x over previous
import functools

import jax
import jax.numpy as jnp
from jax import lax
from jax.experimental import pallas as pl
from jax.experimental.pallas import tpu as pltpu
from jax.experimental.pallas import tpu_sc as plsc

N_NODES = 10000
D_FEAT = 128
N_EDGES = 320000
R_DEC = 2.0
T_DEC = 1.0

_info = plsc.get_sparse_core_info()
NC, NS, L = _info.num_cores, _info.num_subcores, _info.num_lanes
NW = NC * NS
EPW = N_EDGES // NW
B = 80
NB = EPW // B
UNROLL = 8


def _normalize_body(h_ref, o_ref):
    x = h_ref[...]
    n2 = jnp.sum(x * x, axis=1, keepdims=True)
    norm = jnp.sqrt(n2)
    scale = jnp.minimum(jnp.float32(1.0),
                        jnp.float32(1.0) / jnp.maximum(norm, jnp.float32(1e-12)))
    o_ref[...] = x * scale


def _normalize(h):
    rb = 1000
    return pl.pallas_call(
        _normalize_body,
        out_shape=jax.ShapeDtypeStruct((N_NODES, D_FEAT), jnp.float32),
        grid=(N_NODES // rb,),
        in_specs=[pl.BlockSpec((rb, D_FEAT), lambda i: (i, 0))],
        out_specs=pl.BlockSpec((rb, D_FEAT), lambda i: (i, 0)),
    )(h)


@functools.partial(
    pl.kernel,
    out_type=jax.ShapeDtypeStruct((N_EDGES,), jnp.float32),
    mesh=plsc.VectorSubcoreMesh(core_axis_name="c", subcore_axis_name="s"),
    compiler_params=pltpu.CompilerParams(needs_layout_passes=False),
    scratch_types=[
        pltpu.VMEM((EPW,), jnp.int32),
        pltpu.VMEM((EPW,), jnp.int32),
        pltpu.VMEM((EPW,), jnp.float32),
        pltpu.VMEM((B, D_FEAT), jnp.float32),
        pltpu.VMEM((B, D_FEAT), jnp.float32),
        pltpu.VMEM((B, D_FEAT), jnp.float32),
        pltpu.VMEM((B, D_FEAT), jnp.float32),
        pltpu.SemaphoreType.DMA,
        pltpu.SemaphoreType.DMA,
    ],
)
def _sc_decode(h_hbm, ei_hbm, out_hbm, idx_i, idx_j, out_v,
               xi0, xj0, xi1, xj1, sem0, sem1):
    wid = lax.axis_index("s") * NC + lax.axis_index("c")
    base = pl.multiple_of(wid * EPW, 8)
    pltpu.sync_copy(ei_hbm.at[pl.ds(base, EPW)], idx_i)
    pltpu.sync_copy(ei_hbm.at[pl.ds(N_EDGES + base, EPW)], idx_j)

    bufs = ((xi0, xj0, sem0), (xi1, xj1, sem1))

    def start(blk, b):
        xi, xj, sem = bufs[b]
        off = pl.multiple_of(blk * B, 8)
        pltpu.async_copy(h_hbm.at[idx_i.at[pl.ds(off, B)]], xi, sem)
        pltpu.async_copy(h_hbm.at[idx_j.at[pl.ds(off, B)]], xj, sem)

    def wait(blk, b):
        xi, xj, sem = bufs[b]
        off = pl.multiple_of(blk * B, 8)
        pltpu.make_async_copy(h_hbm.at[idx_i.at[pl.ds(off, B)]], xi, sem).wait()
        pltpu.make_async_copy(h_hbm.at[idx_j.at[pl.ds(off, B)]], xj, sem).wait()

    def compute(blk, b):
        xi, xj, _ = bufs[b]
        lane = lax.iota(jnp.int32, L)
        for g in range(B // L):
            rows = jnp.int32(g * L) + lane
            acc0 = (jnp.zeros((L,), jnp.float32), jnp.zeros((L,), jnp.float32))

            def dstep(d, accs):
                acc_a, acc_b = accs
                for u in range(UNROLL):
                    col = (d * UNROLL + u + lane) & jnp.int32(D_FEAT - 1)
                    a = plsc.load_gather(xi, [rows, col])
                    c = plsc.load_gather(xj, [rows, col])
                    t = a - c
                    if u % 2 == 0:
                        acc_a = acc_a + t * t
                    else:
                        acc_b = acc_b + t * t
                return acc_a, acc_b

            acc_a, acc_b = lax.fori_loop(0, D_FEAT // UNROLL, dstep, acc0)
            acc = acc_a + acc_b
            probs = jnp.float32(1.0) / (
                jnp.exp((acc - jnp.float32(R_DEC)) / jnp.float32(T_DEC))
                + jnp.float32(1.0))
            out_v[pl.ds(pl.multiple_of(blk * B + g * L, 8), L)] = probs

    start(0, 0)
    start(1, 1)

    def pair(p, carry):
        g = p * 2
        wait(g, 0)
        compute(g, 0)

        @pl.when(g + 2 < NB)
        def _():
            start(g + 2, 0)

        wait(g + 1, 1)
        compute(g + 1, 1)

        @pl.when(g + 3 < NB)
        def _():
            start(g + 3, 1)

        return carry

    lax.fori_loop(0, (NB - 1) // 2, pair, 0)
    wait(NB - 1, 0)
    compute(NB - 1, 0)

    pltpu.sync_copy(out_v, out_hbm.at[pl.ds(base, EPW)])


def kernel(h, edge_index):
    h_norm = _normalize(h)
    ei_flat = edge_index.astype(jnp.int32).reshape(-1)
    return _sc_decode(h_norm, ei_flat)
